# baseline (speedup 1.0000x reference)
import functools

import jax
import jax.numpy as jnp
from jax import lax
from jax.experimental import pallas as pl
from jax.experimental.pallas import tpu as pltpu
from jax.experimental.pallas import tpu_sc as plsc

B = 1
L = 2048
D_MODEL = 768
D_FF = 2048
E = 8
TOP_K = 2
CAPACITY = int(1.25 * B * L * TOP_K / E)
S = B * L
NROWS = E * CAPACITY
NSLOT = S * TOP_K

_info = plsc.get_sparse_core_info()
NC = _info.num_cores
NS = _info.num_subcores
NW = NC * NS
ROWS_W = NROWS // NW
ROW_CHUNK = ROWS_W // 4
TOK_W = S // NW
SCAT_IT = NSLOT // 16


def _router_body(x_ref, gw_ref, desti_ref, posi_ref, wtsi_ref, xp_ref,
                 aux_ref):
    x = x_ref[...]
    gw = gw_ref[...]
    logits = jax.lax.dot_general(
        x, gw, (((1,), (1,)), ((), ())), preferred_element_type=jnp.float32
    )

    e_iota = jax.lax.broadcasted_iota(jnp.int32, (S, E), 1)
    m1 = jnp.max(logits, axis=1, keepdims=True)
    i1 = jnp.min(jnp.where(logits == m1, e_iota, E), axis=1, keepdims=True)
    masked = jnp.where(e_iota == i1, -jnp.inf, logits)
    m2 = jnp.max(masked, axis=1, keepdims=True)
    i2 = jnp.min(jnp.where(masked == m2, e_iota, E), axis=1, keepdims=True)

    w0 = 1.0 / (1.0 + jnp.exp(m2 - m1))
    w1 = 1.0 - w0

    sel0 = (e_iota == i1).astype(jnp.float32)
    sel1 = (e_iota == i2).astype(jnp.float32)
    cnt = sel0 + sel1
    r_io = jax.lax.broadcasted_iota(jnp.int32, (S, S), 0)
    c_io = jax.lax.broadcasted_iota(jnp.int32, (S, S), 1)
    tri = (r_io > c_io).astype(jnp.float32)
    prefix = jax.lax.dot_general(
        tri, cnt, (((1,), (0,)), ((), ())), preferred_element_type=jnp.float32
    )
    rank0 = jnp.sum(sel0 * prefix, axis=1, keepdims=True).astype(jnp.int32)
    rank1 = jnp.sum(sel1 * prefix, axis=1, keepdims=True).astype(jnp.int32)

    loads = jnp.sum(cnt, axis=0, keepdims=True)
    e_iota1 = jax.lax.broadcasted_iota(jnp.int32, (1, E), 1)
    mn = jnp.min(loads, axis=1, keepdims=True)
    zexp = jnp.min(jnp.where(loads == mn, e_iota1, E), axis=1, keepdims=True)
    zrow = zexp * CAPACITY + (CAPACITY - 1)

    keep0 = rank0 < CAPACITY
    keep1 = rank1 < CAPACITY
    d0 = i1 * CAPACITY + rank0
    d1 = i2 * CAPACITY + rank1
    desti_ref[:, 0:1] = jnp.where(keep0, d0, NROWS)
    desti_ref[:, 1:2] = jnp.where(keep1, d1, NROWS)
    posi_ref[:, 0:1] = jnp.where(keep0, d0, zrow)
    posi_ref[:, 1:2] = jnp.where(keep1, d1, zrow)
    wtsi_ref[:, 0:1] = jnp.where(keep0, w0, 0.0)
    wtsi_ref[:, 1:2] = jnp.where(keep1, w1, 0.0)
    xb = x.astype(jnp.bfloat16)
    lo16 = jax.lax.bitcast_convert_type(xb[:, 0:D_MODEL // 2], jnp.int16)
    hi16 = jax.lax.bitcast_convert_type(xb[:, D_MODEL // 2:D_MODEL], jnp.int16)
    xp_ref[...] = (lo16.astype(jnp.int32) & 0xFFFF) | (hi16.astype(jnp.int32) << 16)

    p = jnp.exp(logits - m1)
    p = p / jnp.sum(p, axis=1, keepdims=True)
    ep = jnp.mean(p, axis=0, keepdims=True)
    aux_ref[...] = jnp.mean((ep - 1.0 / E) ** 2, axis=1, keepdims=True) * E


def _sc_gather_body(dest_hbm, wts_hbm, zi_hbm, zf_hbm, xp_hbm,
                    xg_hbm, wv_hbm,
                    dest_v, wts_v, tok_v, wvl_v, rows_v, sem):
    wid = lax.axis_index("s") * NC + lax.axis_index("c")
    pltpu.sync_copy(dest_hbm, dest_v)
    pltpu.sync_copy(wts_hbm, wts_v)
    pltpu.sync_copy(zi_hbm, tok_v)
    pltpu.sync_copy(zf_hbm, wvl_v)

    def body(j, carry):
        for u in range(8):
            base = j * 128 + u * 16
            dvec = dest_v[pl.ds(base, 16)]
            wvec = wts_v[pl.ds(base, 16)]
            jv = lax.iota(jnp.int32, 16) + base
            tok = lax.bitwise_and(jv, S - 1)
            mask = dvec < NROWS
            plsc.store_scatter(tok_v, [dvec], tok, mask=mask)
            plsc.store_scatter(wvl_v, [dvec], wvec, mask=mask)
        return carry

    lax.fori_loop(0, SCAT_IT // 8, body, 0)

    start = wid * ROWS_W
    pltpu.sync_copy(wvl_v.at[pl.ds(start, ROWS_W)],
                    wv_hbm.at[pl.ds(start, ROWS_W)])
    nchunk = ROWS_W // ROW_CHUNK
    copies = []
    for chunk in range(nchunk):
        cs = start + chunk * ROW_CHUNK
        buf = rows_v.at[chunk % 2]
        copies.append(pltpu.async_copy(
            xp_hbm.at[tok_v.at[pl.ds(cs, ROW_CHUNK)]], buf, sem))
        if chunk >= 1:
            ps = start + (chunk - 1) * ROW_CHUNK
            copies[chunk - 1].wait()
            pltpu.sync_copy(rows_v.at[(chunk - 1) % 2],
                            xg_hbm.at[pl.ds(ps, ROW_CHUNK)])
    copies[nchunk - 1].wait()
    pltpu.sync_copy(rows_v.at[(nchunk - 1) % 2],
                    xg_hbm.at[pl.ds(start + (nchunk - 1) * ROW_CHUNK, ROW_CHUNK)])


def _expert_body(xg_ref, wv_ref, w1_ref, w3_ref, w2_ref, ogw_ref, og_scr,
                 xg_scr):
    f = pl.program_id(1)
    nf = pl.num_programs(1)

    @pl.when(f == 0)
    def _unpack():
        xgp = xg_ref[...]
        xlo = jax.lax.bitcast_convert_type((xgp & 0xFFFF).astype(jnp.int16),
                                           jnp.bfloat16)
        xhi = jax.lax.bitcast_convert_type(
            jax.lax.shift_right_logical(xgp, 16).astype(jnp.int16),
            jnp.bfloat16)
        xg_scr[...] = jnp.concatenate([xlo, xhi], axis=1)

    xg = xg_scr[...]
    w1e = w1_ref[0].astype(jnp.bfloat16)
    w3e = w3_ref[0].astype(jnp.bfloat16)
    w2e = w2_ref[0].astype(jnp.bfloat16)
    h1 = jax.lax.dot_general(xg, w1e, (((1,), (0,)), ((), ())),
                             preferred_element_type=jnp.float32)
    h3 = jax.lax.dot_general(xg, w3e, (((1,), (0,)), ((), ())),
                             preferred_element_type=jnp.float32)
    h = (h1 / (1.0 + jnp.exp(-h1)) * h3).astype(jnp.bfloat16)
    og = jax.lax.dot_general(h, w2e, (((1,), (0,)), ((), ())),
                             preferred_element_type=jnp.float32)

    @pl.when(f == 0)
    def _():
        og_scr[...] = og

    @pl.when(f != 0)
    def _():
        og_scr[...] += og

    @pl.when(f == nf - 1)
    def _():
        ogw = (og_scr[...] * wv_ref[...]).astype(jnp.bfloat16)
        olo = jax.lax.bitcast_convert_type(ogw[:, 0:D_MODEL // 2], jnp.int16)
        ohi = jax.lax.bitcast_convert_type(ogw[:, D_MODEL // 2:D_MODEL],
                                           jnp.int16)
        ogw_ref[...] = ((olo.astype(jnp.int32) & 0xFFFF)
                        | (ohi.astype(jnp.int32) << 16))


def _sc_combine_body(p0_hbm, p1_hbm, ogw_hbm, out_hbm,
                     p0_v, p1_v, r0_v, r1_v, sem):
    wid = lax.axis_index("s") * NC + lax.axis_index("c")
    s0 = wid * TOK_W
    pltpu.sync_copy(p0_hbm.at[pl.ds(s0, TOK_W)], p0_v)
    pltpu.sync_copy(p1_hbm.at[pl.ds(s0, TOK_W)], p1_v)
    pltpu.async_copy(ogw_hbm.at[p0_v], r0_v, sem).wait()
    pltpu.async_copy(ogw_hbm.at[p1_v], r1_v, sem).wait()

    def body(t, carry):
        for c in range(D_MODEL // 32):
            sl = pl.ds(c * 16, 16)
            a = plsc.bitcast(r0_v[t, sl], jnp.bfloat16)
            b = plsc.bitcast(r1_v[t, sl], jnp.bfloat16)
            r0_v[t, sl] = plsc.bitcast(a + b, jnp.int32)
        return carry

    lax.fori_loop(0, TOK_W, body, 0)
    pltpu.sync_copy(r0_v, out_hbm.at[pl.ds(s0, TOK_W)])


_sc_mesh = plsc.VectorSubcoreMesh(core_axis_name="c", subcore_axis_name="s")

_sc_params = pltpu.CompilerParams(needs_layout_passes=False)

_sc_gather = functools.partial(
    pl.kernel,
    mesh=_sc_mesh,
    compiler_params=_sc_params,
    out_type=(
        jax.ShapeDtypeStruct((NROWS, D_MODEL // 2), jnp.int32),
        jax.ShapeDtypeStruct((NROWS,), jnp.float32),
    ),
    scratch_types=[
        pltpu.VMEM((NSLOT,), jnp.int32),
        pltpu.VMEM((NSLOT,), jnp.float32),
        pltpu.VMEM((NROWS,), jnp.int32),
        pltpu.VMEM((NROWS,), jnp.float32),
        pltpu.VMEM((2, ROW_CHUNK, D_MODEL // 2), jnp.int32),
        pltpu.SemaphoreType.DMA,
    ],
)(_sc_gather_body)

_sc_combine = functools.partial(
    pl.kernel,
    mesh=_sc_mesh,
    compiler_params=_sc_params,
    out_type=jax.ShapeDtypeStruct((S, D_MODEL // 2), jnp.int32),
    scratch_types=[
        pltpu.VMEM((TOK_W,), jnp.int32),
        pltpu.VMEM((TOK_W,), jnp.int32),
        pltpu.VMEM((TOK_W, D_MODEL // 2), jnp.int32),
        pltpu.VMEM((TOK_W, D_MODEL // 2), jnp.int32),
        pltpu.SemaphoreType.DMA,
    ],
)(_sc_combine_body)


def _run(x_flat, gate_w, w1, w3, w2):
    desti, posi, wtsi, xp, aux = pl.pallas_call(
        _router_body,
        out_shape=(
            jax.ShapeDtypeStruct((S, TOP_K), jnp.int32),
            jax.ShapeDtypeStruct((S, TOP_K), jnp.int32),
            jax.ShapeDtypeStruct((S, TOP_K), jnp.float32),
            jax.ShapeDtypeStruct((S, D_MODEL // 2), jnp.int32),
            jax.ShapeDtypeStruct((1, 1), jnp.float32),
        ),
    )(x_flat, gate_w)

    dest_flat = desti.T.reshape(NSLOT)
    wts_flat = wtsi.T.reshape(NSLOT)
    pos0 = posi[:, 0]
    pos1 = posi[:, 1]
    zi = jnp.arange(NROWS, dtype=jnp.int32) % S
    zf = jnp.zeros((NROWS,), jnp.float32)

    xg, wv = _sc_gather(dest_flat, wts_flat, zi, zf, xp)

    nf = 2
    f_blk = D_FF // nf
    ogw = pl.pallas_call(
        _expert_body,
        grid=(E, nf),
        in_specs=[
            pl.BlockSpec((CAPACITY, D_MODEL // 2), lambda e, f: (e, 0)),
            pl.BlockSpec((CAPACITY, 1), lambda e, f: (e, 0)),
            pl.BlockSpec((1, D_MODEL, f_blk), lambda e, f: (e, 0, f)),
            pl.BlockSpec((1, D_MODEL, f_blk), lambda e, f: (e, 0, f)),
            pl.BlockSpec((1, f_blk, D_MODEL), lambda e, f: (e, f, 0)),
        ],
        out_specs=pl.BlockSpec((CAPACITY, D_MODEL // 2), lambda e, f: (e, 0)),
        out_shape=jax.ShapeDtypeStruct((NROWS, D_MODEL // 2), jnp.int32),
        scratch_shapes=[
            pltpu.VMEM((CAPACITY, D_MODEL), jnp.float32),
            pltpu.VMEM((CAPACITY, D_MODEL), jnp.bfloat16),
        ],
    )(xg, wv.reshape(NROWS, 1), w1, w3, w2)

    outp = _sc_combine(pos0, pos1, ogw)
    outb = jax.lax.bitcast_convert_type(outp, jnp.bfloat16)
    out = jnp.concatenate([outb[:, :, 0], outb[:, :, 1]],
                          axis=1).astype(jnp.float32)
    return out, aux[0, 0]


@jax.jit
def kernel(x, gate_w, w1, w3, w2):
    x_flat = x.reshape(S, D_MODEL)
    out, aux = _run(x_flat, gate_w, w1, w3, w2)
    return out.reshape(B, L, D_MODEL), aux
